# baseline (speedup 1.0000x reference)
import functools

import jax
import jax.numpy as jnp
from jax.experimental import pallas as pl
from jax.experimental.pallas import tpu as pltpu

F32 = jnp.float32
BF16 = jnp.bfloat16


def _mm_cast_kernel(a_ref, w_ref, o_ref):
    a = a_ref[...].astype(BF16)
    w = w_ref[...].astype(BF16)
    o_ref[...] = jnp.dot(a, w, preferred_element_type=F32).astype(BF16)


def _project(a, w, bm):
    m, k = a.shape
    _, n = w.shape
    return pl.pallas_call(
        _mm_cast_kernel,
        grid=(m // bm,),
        in_specs=[
            pl.BlockSpec((bm, k), lambda i: (i, 0)),
            pl.BlockSpec((k, n), lambda i: (0, 0)),
        ],
        out_specs=pl.BlockSpec((bm, n), lambda i: (i, 0)),
        out_shape=jax.ShapeDtypeStruct((m, n), BF16),
        compiler_params=pltpu.CompilerParams(
            dimension_semantics=("parallel",)),
    )(a, w)


def _pass_a_kernel(adj_ref, u_ref, b1_ref, w2a_ref, za_ref):
    a = adj_ref[...].astype(BF16)
    h = jnp.dot(a, u_ref[...], preferred_element_type=F32)
    h = jnp.maximum(h + b1_ref[...], 0.0).astype(BF16)
    za_ref[...] = jnp.dot(h, w2a_ref[...], preferred_element_type=F32)


def _pass_b_kernel(inc_ref, v_ref, be_ref, w2b_ref, za_ref, z_ref):
    a = inc_ref[...].astype(BF16)
    g = jnp.dot(a, v_ref[...], preferred_element_type=F32)
    g = jnp.maximum(g + be_ref[...], 0.0).astype(BF16)
    z = za_ref[...] + jnp.dot(g, w2b_ref[...], preferred_element_type=F32)
    z_ref[...] = z.astype(BF16)


def _pass_c_kernel(adj_ref, z_ref, b2_ref, o_ref, *, nclass):
    a = adj_ref[...].astype(BF16)
    logits = jnp.dot(a, z_ref[...], preferred_element_type=F32)
    logits = logits + b2_ref[...]
    col = jax.lax.broadcasted_iota(jnp.int32, logits.shape, 1)
    neg = jnp.full_like(logits, -jnp.inf)
    masked = jnp.where(col < nclass, logits, neg)
    m = jnp.max(masked, axis=-1, keepdims=True)
    lse = jnp.log(jnp.sum(jnp.exp(masked - m), axis=-1, keepdims=True))
    out = jnp.where(col < nclass, logits - m - lse, 0.0)
    o_ref[...] = out


def kernel(x, adj, y, inc, W1, b1, We, be, W2, b2):
    n, nfeat = x.shape
    e, efeat = y.shape
    nhid = W1.shape[1]
    nclass = W2.shape[1]
    lanes = 128
    ncp = max(lanes, ((nclass + lanes - 1) // lanes) * lanes)

    u = _project(x, W1, 1024)
    v = _project(y, We, 1024)

    w2a = jnp.zeros((nhid, ncp), BF16).at[:, :nclass].set(W2[:nhid].astype(BF16))
    w2b = jnp.zeros((nhid, ncp), BF16).at[:, :nclass].set(W2[nhid:].astype(BF16))
    b2p = jnp.zeros((1, ncp), F32).at[:, :nclass].set(b2)
    b1r = b1.reshape(1, nhid)
    ber = be.reshape(1, nhid)

    bm_a = 256
    za = pl.pallas_call(
        _pass_a_kernel,
        grid=(n // bm_a,),
        in_specs=[
            pl.BlockSpec((bm_a, n), lambda i: (i, 0)),
            pl.BlockSpec((n, nhid), lambda i: (0, 0)),
            pl.BlockSpec((1, nhid), lambda i: (0, 0)),
            pl.BlockSpec((nhid, ncp), lambda i: (0, 0)),
        ],
        out_specs=pl.BlockSpec((bm_a, ncp), lambda i: (i, 0)),
        out_shape=jax.ShapeDtypeStruct((n, ncp), F32),
        compiler_params=pltpu.CompilerParams(
            dimension_semantics=("parallel",)),
    )(adj, u, b1r, w2a)

    bm_b = 128
    z = pl.pallas_call(
        _pass_b_kernel,
        grid=(n // bm_b,),
        in_specs=[
            pl.BlockSpec((bm_b, e), lambda i: (i, 0)),
            pl.BlockSpec((e, nhid), lambda i: (0, 0)),
            pl.BlockSpec((1, nhid), lambda i: (0, 0)),
            pl.BlockSpec((nhid, ncp), lambda i: (0, 0)),
            pl.BlockSpec((bm_b, ncp), lambda i: (i, 0)),
        ],
        out_specs=pl.BlockSpec((bm_b, ncp), lambda i: (i, 0)),
        out_shape=jax.ShapeDtypeStruct((n, ncp), BF16),
        compiler_params=pltpu.CompilerParams(
            dimension_semantics=("parallel",)),
    )(inc, v, ber, w2b, za)

    bm_c = 256
    outp = pl.pallas_call(
        functools.partial(_pass_c_kernel, nclass=nclass),
        grid=(n // bm_c,),
        in_specs=[
            pl.BlockSpec((bm_c, n), lambda i: (i, 0)),
            pl.BlockSpec((n, ncp), lambda i: (0, 0)),
            pl.BlockSpec((1, ncp), lambda i: (0, 0)),
        ],
        out_specs=pl.BlockSpec((bm_c, ncp), lambda i: (i, 0)),
        out_shape=jax.ShapeDtypeStruct((n, ncp), F32),
        compiler_params=pltpu.CompilerParams(
            dimension_semantics=("parallel",)),
    )(adj, z, b2p)

    return outp[:, :nclass]
